# baseline (speedup 1.0000x reference)
import functools

import jax
import jax.numpy as jnp
from jax import lax
from jax.experimental import pallas as pl
from jax.experimental.pallas import tpu as pltpu
from jax.experimental.pallas import tpu_sc as plsc

N_NODES = 10000
FDIM = 128
E_TOTAL = 320000
NC = 2
NS = 16
NW = NC * NS
CHUNK = 80
CPW = 125
IDXD = 4
ROWS_PER_TILE = 624
ROWS_REM = N_NODES - NS * ROWS_PER_TILE


def _sc_agg_body(feat_hbm, src_hbm, dst_hbm, attr_hbm, out_hbm,
                 src_v, dst_v, attr_v, g0, g1, s0, s1, dsts0, dsts1, acc,
                 sem, sem_g0, sem_g1, sem_s0, sem_s1):
    c = lax.axis_index("c")
    s = lax.axis_index("s")
    wid = s * NC + c

    def zero_body(row, carry):
        for k in range(FDIM // 16):
            g0[row, pl.ds(k * 16, 16)] = jnp.zeros((16,), jnp.float32)
        return carry

    lax.fori_loop(0, CHUNK, zero_body, 0, unroll=False)
    for k in range(7):
        pltpu.async_copy(g0, acc.at[pl.ds(s * ROWS_PER_TILE + k * 80, 80)],
                         sem_s0)
    pltpu.async_copy(g0.at[pl.ds(0, 64)],
                     acc.at[pl.ds(s * ROWS_PER_TILE + 560, 64)], sem_s0)

    @pl.when(s == NS - 1)
    def _():
        pltpu.async_copy(g0.at[pl.ds(0, ROWS_REM)],
                         acc.at[pl.ds(NS * ROWS_PER_TILE, ROWS_REM)], sem_s0)

    for k in range(7):
        pltpu.make_async_copy(
            g0, acc.at[pl.ds(s * ROWS_PER_TILE + k * 80, 80)], sem_s0).wait()
    pltpu.make_async_copy(
        g0.at[pl.ds(0, 64)],
        acc.at[pl.ds(s * ROWS_PER_TILE + 560, 64)], sem_s0).wait()

    @pl.when(s == NS - 1)
    def _():
        pltpu.make_async_copy(
            g0.at[pl.ds(0, ROWS_REM)],
            acc.at[pl.ds(NS * ROWS_PER_TILE, ROWS_REM)], sem_s0).wait()

    plsc.subcore_barrier()

    ch0 = wid * CPW

    def _idx_copies(p, d, n):
        base = (ch0 + 2 * p) * CHUNK
        vb = d * 2 * CHUNK
        return [
            pltpu.make_async_copy(src_hbm.at[pl.ds(base, n * CHUNK)],
                                  src_v.at[pl.ds(vb, n * CHUNK)],
                                  sem.at[d]),
            pltpu.make_async_copy(dst_hbm.at[pl.ds(base, n * CHUNK)],
                                  dst_v.at[pl.ds(vb, n * CHUNK)],
                                  sem.at[d]),
            pltpu.make_async_copy(attr_hbm.at[pl.ds(base, n * CHUNK)],
                                  attr_v.at[pl.ds(vb, n * CHUNK)],
                                  sem.at[d]),
        ]

    def fire_idx(p, d, n=2):
        for cp in _idx_copies(p, d, n):
            cp.start()

    def wait_idx(p, d, n=2):
        for cp in _idx_copies(p, d, n):
            cp.wait()

    HC = CHUNK // 2

    def fire_gather(d, off, g, sem_g):
        vb = d * 2 * CHUNK + off
        pltpu.async_copy(feat_hbm.at[src_v.at[pl.ds(vb, HC)]],
                         g.at[pl.ds(0, HC)], sem_g)
        pltpu.async_copy(feat_hbm.at[src_v.at[pl.ds(vb + HC, HC)]],
                         g.at[pl.ds(HC, HC)], sem_g)

    def wait_gather(d, off, g, sem_g):
        vb = d * 2 * CHUNK + off
        pltpu.make_async_copy(feat_hbm.at[src_v.at[pl.ds(vb, HC)]],
                              g.at[pl.ds(0, HC)], sem_g).wait()
        pltpu.make_async_copy(feat_hbm.at[src_v.at[pl.ds(vb + HC, HC)]],
                              g.at[pl.ds(HC, HC)], sem_g).wait()

    def fire_scatter(g, dsts, sem_s):
        pltpu.async_copy(g, acc.at[dsts], sem_s, add=True)

    def wait_scatter(g, dsts, sem_s):
        pltpu.make_async_copy(g, acc.at[dsts], sem_s).wait()

    def scale_copy(d, off, g, sc, dsts):
        vb = d * 2 * CHUNK + off
        for q in range(CHUNK // 16):
            dsts[pl.ds(q * 16, 16)] = dst_v[pl.ds(vb + q * 16, 16)]

        def scale_body(grp, carry2):
            a16 = attr_v[pl.ds(vb + grp * 16, 16)]
            for j in range(16):
                e = grp * 16 + j
                a = jnp.full((16,), a16[j], jnp.float32)
                for k in range(8):
                    sl = pl.ds(k * 16, 16)
                    sc[e, sl] = g[e, sl] * a
            return carry2

        lax.fori_loop(0, CHUNK // 16, scale_body, 0, unroll=False)

    fire_idx(0, 0)
    fire_idx(1, 1)
    wait_idx(0, 0)
    fire_gather(0, 0, g0, sem_g0)
    fire_gather(0, CHUNK, g1, sem_g1)

    def pair_body(t, carry):
        i0 = 2 * t
        d = t % IDXD
        d1 = (t + 1) % IDXD

        wait_gather(d, 0, g0, sem_g0)

        @pl.when(i0 >= 2)
        def _():
            wait_scatter(s0, dsts0, sem_s0)

        scale_copy(d, 0, g0, s0, dsts0)
        fire_scatter(s0, dsts0, sem_s0)

        @pl.when(i0 + 3 < CPW)
        def _():
            wait_idx(t + 1, d1, 2)

        @pl.when(i0 + 3 == CPW)
        def _():
            wait_idx(t + 1, d1, 1)

        @pl.when(i0 + 2 < CPW)
        def _():
            fire_gather(d1, 0, g0, sem_g0)

        @pl.when(i0 + 5 < CPW)
        def _():
            fire_idx(t + 2, (t + 2) % IDXD, 2)

        @pl.when(i0 + 5 == CPW)
        def _():
            fire_idx(t + 2, (t + 2) % IDXD, 1)

        wait_gather(d, CHUNK, g1, sem_g1)

        @pl.when(i0 >= 1)
        def _():
            wait_scatter(s1, dsts1, sem_s1)

        scale_copy(d, CHUNK, g1, s1, dsts1)
        fire_scatter(s1, dsts1, sem_s1)

        @pl.when(i0 + 3 < CPW)
        def _():
            fire_gather(d1, CHUNK, g1, sem_g1)

        return carry

    lax.fori_loop(0, CPW // 2, pair_body, 0, unroll=False)

    wait_gather((CPW // 2) % IDXD, 0, g0, sem_g0)
    wait_scatter(s0, dsts0, sem_s0)
    scale_copy((CPW // 2) % IDXD, 0, g0, s0, dsts0)
    fire_scatter(s0, dsts0, sem_s0)

    wait_scatter(s1, dsts1, sem_s1)
    wait_scatter(s0, dsts0, sem_s0)

    plsc.subcore_barrier()
    pltpu.sync_copy(acc.at[pl.ds(s * ROWS_PER_TILE, ROWS_PER_TILE)],
                    out_hbm.at[c, pl.ds(s * ROWS_PER_TILE, ROWS_PER_TILE)])

    @pl.when(s == NS - 1)
    def _():
        pltpu.sync_copy(acc.at[pl.ds(NS * ROWS_PER_TILE, ROWS_REM)],
                        out_hbm.at[c, pl.ds(NS * ROWS_PER_TILE, ROWS_REM)])


@jax.jit
def _sc_edge_agg(feat, src, dst, attr):
    mesh = plsc.VectorSubcoreMesh(core_axis_name="c", subcore_axis_name="s")
    kern = pl.kernel(
        _sc_agg_body,
        mesh=mesh,
        out_type=jax.ShapeDtypeStruct((NC, N_NODES, FDIM), jnp.float32),
        scratch_types=[
            pltpu.VMEM((IDXD * 2 * CHUNK,), jnp.int32),
            pltpu.VMEM((IDXD * 2 * CHUNK,), jnp.int32),
            pltpu.VMEM((IDXD * 2 * CHUNK,), jnp.float32),
            pltpu.VMEM((CHUNK, FDIM), jnp.float32),
            pltpu.VMEM((CHUNK, FDIM), jnp.float32),
            pltpu.VMEM((CHUNK, FDIM), jnp.float32),
            pltpu.VMEM((CHUNK, FDIM), jnp.float32),
            pltpu.VMEM((CHUNK,), jnp.int32),
            pltpu.VMEM((CHUNK,), jnp.int32),
            pltpu.VMEM_SHARED((N_NODES, FDIM), jnp.float32),
            pltpu.SemaphoreType.DMA((IDXD,)),
            pltpu.SemaphoreType.DMA,
            pltpu.SemaphoreType.DMA,
            pltpu.SemaphoreType.DMA,
            pltpu.SemaphoreType.DMA,
        ],
    )
    return kern(feat, src, dst, attr)


def _dense1_body(aggp_ref, x_ref, wr_ref, wrt_ref, b_ref, wr2_ref,
                 h1_ref, t_ref):
    agg = aggp_ref[0] + aggp_ref[1]
    h1 = jnp.dot(agg, wr_ref[...], preferred_element_type=jnp.float32)
    h1 += jnp.dot(x_ref[...], wrt_ref[...], preferred_element_type=jnp.float32)
    h1 = jnp.maximum(h1 + b_ref[...], 0.0)
    h1_ref[...] = h1
    t_ref[...] = jnp.dot(h1, wr2_ref[...], preferred_element_type=jnp.float32)


@jax.jit
def _dense1(aggp, x, W_rel1, W_root1, b1, W_rel2):
    R = 1000
    grid = N_NODES // R
    full = lambda shape: pl.BlockSpec(shape, lambda i: (0, 0))
    return pl.pallas_call(
        _dense1_body,
        grid=(grid,),
        in_specs=[
            pl.BlockSpec((NC, R, FDIM), lambda i: (0, i, 0)),
            pl.BlockSpec((R, FDIM), lambda i: (i, 0)),
            full((FDIM, 256)),
            full((FDIM, 256)),
            full((1, 256)),
            full((256, FDIM)),
        ],
        out_specs=[
            pl.BlockSpec((R, 256), lambda i: (i, 0)),
            pl.BlockSpec((R, FDIM), lambda i: (i, 0)),
        ],
        out_shape=[
            jax.ShapeDtypeStruct((N_NODES, 256), jnp.float32),
            jax.ShapeDtypeStruct((N_NODES, FDIM), jnp.float32),
        ],
        compiler_params=pltpu.CompilerParams(
            dimension_semantics=("parallel",)),
    )(aggp, x, W_rel1, W_root1, b1, W_rel2)


def _dense2_body(aggp_ref, h1_ref, wrt2_ref, b2_ref, wl1_ref,
                 bl1_ref, wl2_ref, bl2_ref, h4_ref):
    h2 = aggp_ref[0] + aggp_ref[1]
    h2 += jnp.dot(h1_ref[...], wrt2_ref[...], preferred_element_type=jnp.float32)
    h2 = jnp.maximum(h2 + b2_ref[...], 0.0)
    h3 = jnp.dot(h2, wl1_ref[...], preferred_element_type=jnp.float32)
    h3 = jnp.maximum(h3 + bl1_ref[...], 0.0)
    h4 = jnp.dot(h3, wl2_ref[...], preferred_element_type=jnp.float32)
    h4_ref[...] = h4 + bl2_ref[...]


@jax.jit
def _dense2(aggp, h1, W_root2, b2, Wl1, bl1, Wl2, bl2):
    R = 1000
    grid = N_NODES // R
    full = lambda shape: pl.BlockSpec(shape, lambda i: (0, 0))
    return pl.pallas_call(
        _dense2_body,
        grid=(grid,),
        in_specs=[
            pl.BlockSpec((NC, R, FDIM), lambda i: (0, i, 0)),
            pl.BlockSpec((R, 256), lambda i: (i, 0)),
            full((256, FDIM)),
            full((1, FDIM)),
            full((FDIM, 64)),
            full((1, 64)),
            full((64, 1)),
            full((1, 1)),
        ],
        out_specs=pl.BlockSpec((R, 1), lambda i: (i, 0)),
        out_shape=jax.ShapeDtypeStruct((N_NODES, 1), jnp.float32),
        compiler_params=pltpu.CompilerParams(
            dimension_semantics=("parallel",)),
    )(aggp, h1, W_root2, b2, Wl1, bl1, Wl2, bl2)


def _final_body(h_ref, wlast_ref, blast_ref, out_ref):
    o = jnp.dot(h_ref[...], wlast_ref[...], preferred_element_type=jnp.float32)
    out_ref[...] = jax.nn.sigmoid(o + blast_ref[...])


@jax.jit
def _final(H, Wlast, blast):
    return pl.pallas_call(
        _final_body,
        out_shape=jax.ShapeDtypeStruct((100, 1), jnp.float32),
    )(H, Wlast, blast)


def kernel(x, edge_index, edge_attribute, W_rel1, W_root1, b1, W_rel2,
           W_root2, b2, Wl1, bl1, Wl2, bl2, Wlast, blast):
    src = edge_index[0]
    dst = edge_index[1]

    aggp1 = _sc_edge_agg(x, src, dst, edge_attribute)
    h1, t = _dense1(aggp1, x, W_rel1, W_root1, b1.reshape(1, 256), W_rel2)
    aggp2 = _sc_edge_agg(t, src, dst, edge_attribute)
    h4 = _dense2(aggp2, h1, W_root2, b2.reshape(1, FDIM),
                 Wl1, bl1.reshape(1, 64), Wl2, bl2.reshape(1, 1))
    H = h4.reshape(100, 100)
    return _final(H, Wlast, blast.reshape(1, 1))


# speedup vs baseline: 1.0068x; 1.0068x over previous
import functools

import jax
import jax.numpy as jnp
from jax import lax
from jax.experimental import pallas as pl
from jax.experimental.pallas import tpu as pltpu
from jax.experimental.pallas import tpu_sc as plsc

N_NODES = 10000
FDIM = 128
E_TOTAL = 320000
NC = 2
NS = 16
NW = NC * NS
CHUNK = 80
CPW = 125
IDXD = 4
ROWS_PER_TILE = 624
ROWS_REM = N_NODES - NS * ROWS_PER_TILE


def _sc_agg_body(feat_hbm, src_hbm, dst_hbm, attr_hbm, out_hbm,
                 src_v, dst_v, attr_v, g0, g1, s0, s1, dsts0, dsts1, acc,
                 sem, sem_g0, sem_g1, sem_s0, sem_s1):
    c = lax.axis_index("c")
    s = lax.axis_index("s")
    wid = s * NC + c

    def zero_body(row, carry):
        for k in range(FDIM // 16):
            g0[row, pl.ds(k * 16, 16)] = jnp.zeros((16,), jnp.float32)
        return carry

    lax.fori_loop(0, CHUNK, zero_body, 0, unroll=False)
    for k in range(7):
        pltpu.async_copy(g0, acc.at[pl.ds(s * ROWS_PER_TILE + k * 80, 80)],
                         sem_s0)
    pltpu.async_copy(g0.at[pl.ds(0, 64)],
                     acc.at[pl.ds(s * ROWS_PER_TILE + 560, 64)], sem_s0)

    @pl.when(s == NS - 1)
    def _():
        pltpu.async_copy(g0.at[pl.ds(0, ROWS_REM)],
                         acc.at[pl.ds(NS * ROWS_PER_TILE, ROWS_REM)], sem_s0)

    for k in range(7):
        pltpu.make_async_copy(
            g0, acc.at[pl.ds(s * ROWS_PER_TILE + k * 80, 80)], sem_s0).wait()
    pltpu.make_async_copy(
        g0.at[pl.ds(0, 64)],
        acc.at[pl.ds(s * ROWS_PER_TILE + 560, 64)], sem_s0).wait()

    @pl.when(s == NS - 1)
    def _():
        pltpu.make_async_copy(
            g0.at[pl.ds(0, ROWS_REM)],
            acc.at[pl.ds(NS * ROWS_PER_TILE, ROWS_REM)], sem_s0).wait()

    plsc.subcore_barrier()

    ch0 = wid * CPW

    def _idx_copies(p, d, n):
        base = (ch0 + 2 * p) * CHUNK
        vb = d * 2 * CHUNK
        return [
            pltpu.make_async_copy(src_hbm.at[pl.ds(base, n * CHUNK)],
                                  src_v.at[pl.ds(vb, n * CHUNK)],
                                  sem.at[d]),
            pltpu.make_async_copy(dst_hbm.at[pl.ds(base, n * CHUNK)],
                                  dst_v.at[pl.ds(vb, n * CHUNK)],
                                  sem.at[d]),
            pltpu.make_async_copy(attr_hbm.at[pl.ds(base, n * CHUNK)],
                                  attr_v.at[pl.ds(vb, n * CHUNK)],
                                  sem.at[d]),
        ]

    def fire_idx(p, d, n=2):
        for cp in _idx_copies(p, d, n):
            cp.start()

    def wait_idx(p, d, n=2):
        for cp in _idx_copies(p, d, n):
            cp.wait()

    def fire_gather(d, off, g, sem_g):
        pltpu.async_copy(
            feat_hbm.at[src_v.at[pl.ds(d * 2 * CHUNK + off, CHUNK)]], g,
            sem_g)

    def wait_gather(d, off, g, sem_g):
        pltpu.make_async_copy(
            feat_hbm.at[src_v.at[pl.ds(d * 2 * CHUNK + off, CHUNK)]], g,
            sem_g).wait()

    def fire_scatter(g, dsts, sem_s):
        pltpu.async_copy(g, acc.at[dsts], sem_s, add=True)

    def wait_scatter(g, dsts, sem_s):
        pltpu.make_async_copy(g, acc.at[dsts], sem_s).wait()

    def scale_copy(d, off, g, sc, dsts):
        vb = d * 2 * CHUNK + off
        for q in range(CHUNK // 16):
            dsts[pl.ds(q * 16, 16)] = dst_v[pl.ds(vb + q * 16, 16)]

        def scale_body(grp, carry2):
            a16 = attr_v[pl.ds(vb + grp * 16, 16)]
            for j in range(16):
                e = grp * 16 + j
                a = jnp.full((16,), a16[j], jnp.float32)
                for k in range(8):
                    sl = pl.ds(k * 16, 16)
                    sc[e, sl] = g[e, sl] * a
            return carry2

        lax.fori_loop(0, CHUNK // 16, scale_body, 0, unroll=False)

    fire_idx(0, 0)
    fire_idx(1, 1)
    wait_idx(0, 0)
    fire_gather(0, 0, g0, sem_g0)
    fire_gather(0, CHUNK, g1, sem_g1)

    def pair_body(t, carry):
        i0 = 2 * t
        d = t % IDXD
        d1 = (t + 1) % IDXD

        wait_gather(d, 0, g0, sem_g0)

        @pl.when(i0 >= 2)
        def _():
            wait_scatter(s0, dsts0, sem_s0)

        scale_copy(d, 0, g0, s0, dsts0)
        fire_scatter(s0, dsts0, sem_s0)

        @pl.when(i0 + 3 < CPW)
        def _():
            wait_idx(t + 1, d1, 2)

        @pl.when(i0 + 3 == CPW)
        def _():
            wait_idx(t + 1, d1, 1)

        @pl.when(i0 + 2 < CPW)
        def _():
            fire_gather(d1, 0, g0, sem_g0)

        @pl.when(i0 + 5 < CPW)
        def _():
            fire_idx(t + 2, (t + 2) % IDXD, 2)

        @pl.when(i0 + 5 == CPW)
        def _():
            fire_idx(t + 2, (t + 2) % IDXD, 1)

        wait_gather(d, CHUNK, g1, sem_g1)

        @pl.when(i0 >= 1)
        def _():
            wait_scatter(s1, dsts1, sem_s1)

        scale_copy(d, CHUNK, g1, s1, dsts1)
        fire_scatter(s1, dsts1, sem_s1)

        @pl.when(i0 + 3 < CPW)
        def _():
            fire_gather(d1, CHUNK, g1, sem_g1)

        return carry

    lax.fori_loop(0, CPW // 2, pair_body, 0, unroll=False)

    wait_gather((CPW // 2) % IDXD, 0, g0, sem_g0)
    wait_scatter(s0, dsts0, sem_s0)
    scale_copy((CPW // 2) % IDXD, 0, g0, s0, dsts0)
    fire_scatter(s0, dsts0, sem_s0)

    wait_scatter(s1, dsts1, sem_s1)
    wait_scatter(s0, dsts0, sem_s0)

    plsc.subcore_barrier()
    pltpu.sync_copy(acc.at[pl.ds(s * ROWS_PER_TILE, ROWS_PER_TILE)],
                    out_hbm.at[c, pl.ds(s * ROWS_PER_TILE, ROWS_PER_TILE)])

    @pl.when(s == NS - 1)
    def _():
        pltpu.sync_copy(acc.at[pl.ds(NS * ROWS_PER_TILE, ROWS_REM)],
                        out_hbm.at[c, pl.ds(NS * ROWS_PER_TILE, ROWS_REM)])


@jax.jit
def _sc_edge_agg(feat, src, dst, attr):
    mesh = plsc.VectorSubcoreMesh(core_axis_name="c", subcore_axis_name="s")
    kern = pl.kernel(
        _sc_agg_body,
        mesh=mesh,
        out_type=jax.ShapeDtypeStruct((NC, N_NODES, FDIM), jnp.float32),
        scratch_types=[
            pltpu.VMEM((IDXD * 2 * CHUNK,), jnp.int32),
            pltpu.VMEM((IDXD * 2 * CHUNK,), jnp.int32),
            pltpu.VMEM((IDXD * 2 * CHUNK,), jnp.float32),
            pltpu.VMEM((CHUNK, FDIM), jnp.float32),
            pltpu.VMEM((CHUNK, FDIM), jnp.float32),
            pltpu.VMEM((CHUNK, FDIM), jnp.float32),
            pltpu.VMEM((CHUNK, FDIM), jnp.float32),
            pltpu.VMEM((CHUNK,), jnp.int32),
            pltpu.VMEM((CHUNK,), jnp.int32),
            pltpu.VMEM_SHARED((N_NODES, FDIM), jnp.float32),
            pltpu.SemaphoreType.DMA((IDXD,)),
            pltpu.SemaphoreType.DMA,
            pltpu.SemaphoreType.DMA,
            pltpu.SemaphoreType.DMA,
            pltpu.SemaphoreType.DMA,
        ],
    )
    return kern(feat, src, dst, attr)


def _rootmul_body(x_ref, w_ref, b_ref, o_ref):
    o = jnp.dot(x_ref[...], w_ref[...], preferred_element_type=jnp.float32)
    o_ref[...] = o + b_ref[...]


@jax.jit
def _rootmul(x, W, b):
    R = 1000
    grid = N_NODES // R
    din, dout = W.shape
    return pl.pallas_call(
        _rootmul_body,
        grid=(grid,),
        in_specs=[
            pl.BlockSpec((R, din), lambda i: (i, 0)),
            pl.BlockSpec((din, dout), lambda i: (0, 0)),
            pl.BlockSpec((1, dout), lambda i: (0, 0)),
        ],
        out_specs=pl.BlockSpec((R, dout), lambda i: (i, 0)),
        out_shape=jax.ShapeDtypeStruct((N_NODES, dout), jnp.float32),
        compiler_params=pltpu.CompilerParams(
            dimension_semantics=("parallel",)),
    )(x, W, b)


def _dense1_body(aggp_ref, xr_ref, wr_ref, wr2_ref, h1_ref, t_ref):
    agg = aggp_ref[0] + aggp_ref[1]
    h1 = jnp.dot(agg, wr_ref[...], preferred_element_type=jnp.float32)
    h1 = jnp.maximum(h1 + xr_ref[...], 0.0)
    h1_ref[...] = h1
    t_ref[...] = jnp.dot(h1, wr2_ref[...], preferred_element_type=jnp.float32)


@jax.jit
def _dense1(aggp, xr, W_rel1, W_rel2):
    R = 1000
    grid = N_NODES // R
    full = lambda shape: pl.BlockSpec(shape, lambda i: (0, 0))
    return pl.pallas_call(
        _dense1_body,
        grid=(grid,),
        in_specs=[
            pl.BlockSpec((NC, R, FDIM), lambda i: (0, i, 0)),
            pl.BlockSpec((R, 256), lambda i: (i, 0)),
            full((FDIM, 256)),
            full((256, FDIM)),
        ],
        out_specs=[
            pl.BlockSpec((R, 256), lambda i: (i, 0)),
            pl.BlockSpec((R, FDIM), lambda i: (i, 0)),
        ],
        out_shape=[
            jax.ShapeDtypeStruct((N_NODES, 256), jnp.float32),
            jax.ShapeDtypeStruct((N_NODES, FDIM), jnp.float32),
        ],
        compiler_params=pltpu.CompilerParams(
            dimension_semantics=("parallel",)),
    )(aggp, xr, W_rel1, W_rel2)


def _dense2_body(aggp_ref, hr_ref, wl1_ref, bl1_ref, wl2_ref, bl2_ref,
                 h4_ref):
    h2 = aggp_ref[0] + aggp_ref[1]
    h2 = jnp.maximum(h2 + hr_ref[...], 0.0)
    h3 = jnp.dot(h2, wl1_ref[...], preferred_element_type=jnp.float32)
    h3 = jnp.maximum(h3 + bl1_ref[...], 0.0)
    h4 = jnp.dot(h3, wl2_ref[...], preferred_element_type=jnp.float32)
    h4_ref[...] = h4 + bl2_ref[...]


@jax.jit
def _dense2(aggp, hr, Wl1, bl1, Wl2, bl2):
    R = 1000
    grid = N_NODES // R
    full = lambda shape: pl.BlockSpec(shape, lambda i: (0, 0))
    return pl.pallas_call(
        _dense2_body,
        grid=(grid,),
        in_specs=[
            pl.BlockSpec((NC, R, FDIM), lambda i: (0, i, 0)),
            pl.BlockSpec((R, FDIM), lambda i: (i, 0)),
            full((FDIM, 64)),
            full((1, 64)),
            full((64, 1)),
            full((1, 1)),
        ],
        out_specs=pl.BlockSpec((R, 1), lambda i: (i, 0)),
        out_shape=jax.ShapeDtypeStruct((N_NODES, 1), jnp.float32),
        compiler_params=pltpu.CompilerParams(
            dimension_semantics=("parallel",)),
    )(aggp, hr, Wl1, bl1, Wl2, bl2)


def _final_body(h_ref, wlast_ref, blast_ref, out_ref):
    o = jnp.dot(h_ref[...], wlast_ref[...], preferred_element_type=jnp.float32)
    out_ref[...] = jax.nn.sigmoid(o + blast_ref[...])


@jax.jit
def _final(H, Wlast, blast):
    return pl.pallas_call(
        _final_body,
        out_shape=jax.ShapeDtypeStruct((100, 1), jnp.float32),
    )(H, Wlast, blast)


def kernel(x, edge_index, edge_attribute, W_rel1, W_root1, b1, W_rel2,
           W_root2, b2, Wl1, bl1, Wl2, bl2, Wlast, blast):
    src = edge_index[0]
    dst = edge_index[1]

    aggp1 = _sc_edge_agg(x, src, dst, edge_attribute)
    xr = _rootmul(x, W_root1, b1.reshape(1, 256))
    h1, t = _dense1(aggp1, xr, W_rel1, W_rel2)
    aggp2 = _sc_edge_agg(t, src, dst, edge_attribute)
    hr2 = _rootmul(h1, W_root2, b2.reshape(1, FDIM))
    h4 = _dense2(aggp2, hr2, Wl1, bl1.reshape(1, 64), Wl2, bl2.reshape(1, 1))
    H = h4.reshape(100, 100)
    return _final(H, Wlast, blast.reshape(1, 1))


# speedup vs baseline: 1.0108x; 1.0040x over previous
import jax
import jax.numpy as jnp
from jax import lax
from jax.experimental import pallas as pl
from jax.experimental.pallas import tpu as pltpu
from jax.experimental.pallas import tpu_sc as plsc

N_NODES = 10000
FDIM = 128
E_TOTAL = 320000
NC = 2
NS = 16
NW = NC * NS
CHUNK = 80
CPW = 125
IDXD = 4
ROWS_PER_TILE = 624
ROWS_REM = N_NODES - NS * ROWS_PER_TILE


def _sc_agg_body(feat_hbm, src_hbm, dst_hbm, attr_hbm, out_hbm,
                 src_v, dst_v, attr_v, g0, g1, s0, s1, dsts0, dsts1, acc,
                 sem, sem_g0, sem_g1, sem_s0, sem_s1):
    c = lax.axis_index("c")
    s = lax.axis_index("s")
    wid = s * NC + c

    def zero_body(row, carry):
        for k in range(FDIM // 16):
            g0[row, pl.ds(k * 16, 16)] = jnp.zeros((16,), jnp.float32)
        return carry

    lax.fori_loop(0, CHUNK, zero_body, 0, unroll=False)
    for k in range(7):
        pltpu.async_copy(g0, acc.at[pl.ds(s * ROWS_PER_TILE + k * 80, 80)],
                         sem_s0)
    pltpu.async_copy(g0.at[pl.ds(0, 64)],
                     acc.at[pl.ds(s * ROWS_PER_TILE + 560, 64)], sem_s0)

    @pl.when(s == NS - 1)
    def _():
        pltpu.async_copy(g0.at[pl.ds(0, ROWS_REM)],
                         acc.at[pl.ds(NS * ROWS_PER_TILE, ROWS_REM)], sem_s0)

    for k in range(7):
        pltpu.make_async_copy(
            g0, acc.at[pl.ds(s * ROWS_PER_TILE + k * 80, 80)], sem_s0).wait()
    pltpu.make_async_copy(
        g0.at[pl.ds(0, 64)],
        acc.at[pl.ds(s * ROWS_PER_TILE + 560, 64)], sem_s0).wait()

    @pl.when(s == NS - 1)
    def _():
        pltpu.make_async_copy(
            g0.at[pl.ds(0, ROWS_REM)],
            acc.at[pl.ds(NS * ROWS_PER_TILE, ROWS_REM)], sem_s0).wait()

    plsc.subcore_barrier()

    ch0 = wid * CPW

    def _idx_copies(p, d, n):
        base = (ch0 + 2 * p) * CHUNK
        vb = d * 2 * CHUNK
        return [
            pltpu.make_async_copy(src_hbm.at[pl.ds(base, n * CHUNK)],
                                  src_v.at[pl.ds(vb, n * CHUNK)],
                                  sem.at[d]),
            pltpu.make_async_copy(dst_hbm.at[pl.ds(base, n * CHUNK)],
                                  dst_v.at[pl.ds(vb, n * CHUNK)],
                                  sem.at[d]),
            pltpu.make_async_copy(attr_hbm.at[pl.ds(base, n * CHUNK)],
                                  attr_v.at[pl.ds(vb, n * CHUNK)],
                                  sem.at[d]),
        ]

    def fire_idx(p, d, n=2):
        for cp in _idx_copies(p, d, n):
            cp.start()

    def wait_idx(p, d, n=2):
        for cp in _idx_copies(p, d, n):
            cp.wait()

    def fire_gather(d, off, g, sem_g):
        pltpu.async_copy(
            feat_hbm.at[src_v.at[pl.ds(d * 2 * CHUNK + off, CHUNK)]], g,
            sem_g)

    def wait_gather(d, off, g, sem_g):
        pltpu.make_async_copy(
            feat_hbm.at[src_v.at[pl.ds(d * 2 * CHUNK + off, CHUNK)]], g,
            sem_g).wait()

    def fire_scatter(g, dsts, sem_s):
        pltpu.async_copy(g, acc.at[dsts], sem_s, add=True)

    def wait_scatter(g, dsts, sem_s):
        pltpu.make_async_copy(g, acc.at[dsts], sem_s).wait()

    def scale_copy(d, off, g, sc, dsts):
        vb = d * 2 * CHUNK + off
        for q in range(CHUNK // 16):
            dsts[pl.ds(q * 16, 16)] = dst_v[pl.ds(vb + q * 16, 16)]

        def scale_body(grp, carry2):
            a16 = attr_v[pl.ds(vb + grp * 16, 16)]
            for j in range(16):
                e = grp * 16 + j
                a = jnp.full((16,), a16[j], jnp.float32)
                for k in range(8):
                    sl = pl.ds(k * 16, 16)
                    sc[e, sl] = g[e, sl] * a
            return carry2

        lax.fori_loop(0, CHUNK // 16, scale_body, 0, unroll=False)

    fire_idx(0, 0)
    fire_idx(1, 1)
    wait_idx(0, 0)
    fire_gather(0, 0, g0, sem_g0)
    fire_gather(0, CHUNK, g1, sem_g1)

    def pair_body(t, carry):
        i0 = 2 * t
        d = t % IDXD
        d1 = (t + 1) % IDXD

        wait_gather(d, 0, g0, sem_g0)

        @pl.when(i0 >= 2)
        def _():
            wait_scatter(s0, dsts0, sem_s0)

        scale_copy(d, 0, g0, s0, dsts0)
        fire_scatter(s0, dsts0, sem_s0)

        @pl.when(i0 + 3 < CPW)
        def _():
            wait_idx(t + 1, d1, 2)

        @pl.when(i0 + 3 == CPW)
        def _():
            wait_idx(t + 1, d1, 1)

        @pl.when(i0 + 2 < CPW)
        def _():
            fire_gather(d1, 0, g0, sem_g0)

        @pl.when(i0 + 5 < CPW)
        def _():
            fire_idx(t + 2, (t + 2) % IDXD, 2)

        @pl.when(i0 + 5 == CPW)
        def _():
            fire_idx(t + 2, (t + 2) % IDXD, 1)

        wait_gather(d, CHUNK, g1, sem_g1)

        @pl.when(i0 >= 1)
        def _():
            wait_scatter(s1, dsts1, sem_s1)

        scale_copy(d, CHUNK, g1, s1, dsts1)
        fire_scatter(s1, dsts1, sem_s1)

        @pl.when(i0 + 3 < CPW)
        def _():
            fire_gather(d1, CHUNK, g1, sem_g1)

        return carry

    lax.fori_loop(0, CPW // 2, pair_body, 0, unroll=False)

    wait_gather((CPW // 2) % IDXD, 0, g0, sem_g0)
    wait_scatter(s0, dsts0, sem_s0)
    scale_copy((CPW // 2) % IDXD, 0, g0, s0, dsts0)
    fire_scatter(s0, dsts0, sem_s0)

    wait_scatter(s1, dsts1, sem_s1)
    wait_scatter(s0, dsts0, sem_s0)

    plsc.subcore_barrier()
    pltpu.sync_copy(acc.at[pl.ds(s * ROWS_PER_TILE, ROWS_PER_TILE)],
                    out_hbm.at[c, pl.ds(s * ROWS_PER_TILE, ROWS_PER_TILE)])

    @pl.when(s == NS - 1)
    def _():
        pltpu.sync_copy(acc.at[pl.ds(NS * ROWS_PER_TILE, ROWS_REM)],
                        out_hbm.at[c, pl.ds(NS * ROWS_PER_TILE, ROWS_REM)])


@jax.jit
def _sc_edge_agg(feat, src, dst, attr):
    mesh = plsc.VectorSubcoreMesh(core_axis_name="c", subcore_axis_name="s")
    kern = pl.kernel(
        _sc_agg_body,
        mesh=mesh,
        out_type=jax.ShapeDtypeStruct((NC, N_NODES, FDIM), jnp.float32),
        scratch_types=[
            pltpu.VMEM((IDXD * 2 * CHUNK,), jnp.int32),
            pltpu.VMEM((IDXD * 2 * CHUNK,), jnp.int32),
            pltpu.VMEM((IDXD * 2 * CHUNK,), jnp.float32),
            pltpu.VMEM((CHUNK, FDIM), jnp.float32),
            pltpu.VMEM((CHUNK, FDIM), jnp.float32),
            pltpu.VMEM((CHUNK, FDIM), jnp.float32),
            pltpu.VMEM((CHUNK, FDIM), jnp.float32),
            pltpu.VMEM((CHUNK,), jnp.int32),
            pltpu.VMEM((CHUNK,), jnp.int32),
            pltpu.VMEM_SHARED((N_NODES, FDIM), jnp.float32),
            pltpu.SemaphoreType.DMA((IDXD,)),
            pltpu.SemaphoreType.DMA,
            pltpu.SemaphoreType.DMA,
            pltpu.SemaphoreType.DMA,
            pltpu.SemaphoreType.DMA,
        ],
    )
    return kern(feat, src, dst, attr)


def _rootmul_body(x_ref, w_ref, b_ref, o_ref):
    o = jnp.dot(x_ref[...], w_ref[...], preferred_element_type=jnp.float32)
    o_ref[...] = o + b_ref[...]


@jax.jit
def _rootmul(x, W, b):
    R = 1000
    grid = N_NODES // R
    din, dout = W.shape
    return pl.pallas_call(
        _rootmul_body,
        grid=(grid,),
        in_specs=[
            pl.BlockSpec((R, din), lambda i: (i, 0)),
            pl.BlockSpec((din, dout), lambda i: (0, 0)),
            pl.BlockSpec((1, dout), lambda i: (0, 0)),
        ],
        out_specs=pl.BlockSpec((R, dout), lambda i: (i, 0)),
        out_shape=jax.ShapeDtypeStruct((N_NODES, dout), jnp.float32),
        compiler_params=pltpu.CompilerParams(
            dimension_semantics=("parallel",)),
    )(x, W, b)


def _dense1_body(aggp_ref, xr_ref, wr_ref, wr2_ref, h1_ref, t_ref):
    agg = aggp_ref[0] + aggp_ref[1]
    h1 = jnp.dot(agg, wr_ref[...], preferred_element_type=jnp.float32)
    h1 = jnp.maximum(h1 + xr_ref[...], 0.0)
    h1_ref[...] = h1
    t_ref[...] = jnp.dot(h1, wr2_ref[...], preferred_element_type=jnp.float32)


@jax.jit
def _dense1(aggp, xr, W_rel1, W_rel2):
    R = 1000
    grid = N_NODES // R
    full = lambda shape: pl.BlockSpec(shape, lambda i: (0, 0))
    return pl.pallas_call(
        _dense1_body,
        grid=(grid,),
        in_specs=[
            pl.BlockSpec((NC, R, FDIM), lambda i: (0, i, 0)),
            pl.BlockSpec((R, 256), lambda i: (i, 0)),
            full((FDIM, 256)),
            full((256, FDIM)),
        ],
        out_specs=[
            pl.BlockSpec((R, 256), lambda i: (i, 0)),
            pl.BlockSpec((R, FDIM), lambda i: (i, 0)),
        ],
        out_shape=[
            jax.ShapeDtypeStruct((N_NODES, 256), jnp.float32),
            jax.ShapeDtypeStruct((N_NODES, FDIM), jnp.float32),
        ],
        compiler_params=pltpu.CompilerParams(
            dimension_semantics=("parallel",)),
    )(aggp, xr, W_rel1, W_rel2)


def _dense2_body(aggp_ref, hr_ref, wl1_ref, bl1_ref, wl2_ref, bl2_ref,
                 h4_ref):
    h2 = aggp_ref[0] + aggp_ref[1]
    h2 = jnp.maximum(h2 + hr_ref[...], 0.0)
    h3 = jnp.dot(h2, wl1_ref[...], preferred_element_type=jnp.float32)
    h3 = jnp.maximum(h3 + bl1_ref[...], 0.0)
    h4 = jnp.dot(h3, wl2_ref[...], preferred_element_type=jnp.float32)
    h4_ref[...] = h4 + bl2_ref[...]


@jax.jit
def _dense2(aggp, hr, Wl1, bl1, Wl2, bl2):
    R = 1000
    grid = N_NODES // R
    full = lambda shape: pl.BlockSpec(shape, lambda i: (0, 0))
    return pl.pallas_call(
        _dense2_body,
        grid=(grid,),
        in_specs=[
            pl.BlockSpec((NC, R, FDIM), lambda i: (0, i, 0)),
            pl.BlockSpec((R, FDIM), lambda i: (i, 0)),
            full((FDIM, 64)),
            full((1, 64)),
            full((64, 1)),
            full((1, 1)),
        ],
        out_specs=pl.BlockSpec((R, 1), lambda i: (i, 0)),
        out_shape=jax.ShapeDtypeStruct((N_NODES, 1), jnp.float32),
        compiler_params=pltpu.CompilerParams(
            dimension_semantics=("parallel",)),
    )(aggp, hr, Wl1, bl1, Wl2, bl2)


def _final_body(h_ref, wlast_ref, blast_ref, out_ref):
    o = jnp.dot(h_ref[...], wlast_ref[...], preferred_element_type=jnp.float32)
    out_ref[...] = jax.nn.sigmoid(o + blast_ref[...])


@jax.jit
def _final(H, Wlast, blast):
    return pl.pallas_call(
        _final_body,
        out_shape=jax.ShapeDtypeStruct((100, 1), jnp.float32),
    )(H, Wlast, blast)


def kernel(x, edge_index, edge_attribute, W_rel1, W_root1, b1, W_rel2,
           W_root2, b2, Wl1, bl1, Wl2, bl2, Wlast, blast):
    src = edge_index[0]
    dst = edge_index[1]

    aggp1 = _sc_edge_agg(x, src, dst, edge_attribute)
    xr = _rootmul(x, W_root1, b1.reshape(1, 256))
    h1, t = _dense1(aggp1, xr, W_rel1, W_rel2)
    aggp2 = _sc_edge_agg(t, src, dst, edge_attribute)
    hr2 = _rootmul(h1, W_root2, b2.reshape(1, FDIM))
    h4 = _dense2(aggp2, hr2, Wl1, bl1.reshape(1, 64), Wl2, bl2.reshape(1, 1))
    H = h4.reshape(100, 100)
    return _final(H, Wlast, blast.reshape(1, 1))
